# Initial kernel scaffold; baseline (speedup 1.0000x reference)
import jax
import jax.numpy as jnp
from jax.experimental import pallas as pl


def kernel(x, edge_index, rel_emb, W_node, b_node, W_src, b_src, W_rel, W_res, b_res, res_w, w_cross, W_prop, b_prop):
    raise NotImplementedError("write your pallas kernel here")


# speedup vs baseline: 53.6072x; 53.6072x over previous
import functools

import jax
import jax.numpy as jnp
from jax import lax
from jax.experimental import pallas as pl
from jax.experimental.pallas import tpu as pltpu
from jax.experimental.pallas import tpu_sc as plsc

N = 10000
E = 320000
D_IN = 128
H = 8
D_H = 16
HD = H * D_H
SW = 144

NC = 2
NS = 16
C = 80
EPT = E // (NC * NS)
NCH = EPT // C
RPT = N // NS


def _pre_body(x_r, Wn_r, bn_r, Ws_r, bs_r, Wr_r, Wres_r, bres_r, re_r,
              Wp_r, bp_r, S_r, D_r, res_r, smax_r, relo_r):
    i = pl.program_id(0)
    xb = x_r[...]
    h = jnp.dot(xb, Wn_r[...], preferred_element_type=jnp.float32) + bn_r[...]
    s = jnp.dot(h, Ws_r[...], preferred_element_type=jnp.float32) + bs_r[...]
    S_r[:, 0:HD] = s
    res_r[...] = jnp.dot(xb, Wres_r[...], preferred_element_type=jnp.float32) + bres_r[...]
    r = jnp.dot(re_r[...], Wr_r[...], preferred_element_type=jnp.float32)
    es_l, ed_l = [], []
    for hh in range(H):
        rs = r[:, 32 * hh:32 * hh + 16]
        rd = r[:, 32 * hh + 16:32 * hh + 32]
        es_l.append(jnp.sum(s[:, 16 * hh:16 * hh + 16] * rs, axis=1, keepdims=True))
        ed_l.append(jnp.sum(h[:, 16 * hh:16 * hh + 16] * rd, axis=1, keepdims=True))
    es = jnp.concatenate(es_l, axis=1)
    ed = jnp.concatenate(ed_l, axis=1)
    S_r[:, HD:HD + H] = es
    S_r[:, HD + H:SW] = es
    D_r[:, 0:H] = ed
    D_r[:, H:2 * H] = ed

    @pl.when(i == 0)
    def _():
        smax_r[...] = jnp.full((1, H), -1e30, jnp.float32)
        relo_r[...] = jnp.dot(re_r[...], Wp_r[...],
                              preferred_element_type=jnp.float32) + bp_r[...]

    smax_r[...] = jnp.maximum(smax_r[...], jnp.max(es, axis=0, keepdims=True))


def _pre(x, W_node, b_node, W_src, b_src, W_rel, W_res, b_res, rel_emb2,
         W_prop, b_prop):
    BN = 1000
    grid = (N // BN,)
    full = lambda shp: pl.BlockSpec(shp, lambda i: (0, 0))
    return pl.pallas_call(
        _pre_body,
        grid=grid,
        in_specs=[
            pl.BlockSpec((BN, D_IN), lambda i: (i, 0)),
            full((D_IN, HD)), full((1, HD)),
            full((HD, HD)), full((1, HD)),
            full((64, 256)),
            full((D_IN, HD)), full((1, HD)),
            full((1, 64)),
            full((64, HD)), full((1, HD)),
        ],
        out_specs=[
            pl.BlockSpec((BN, SW), lambda i: (i, 0)),
            pl.BlockSpec((BN, 2 * H), lambda i: (i, 0)),
            pl.BlockSpec((BN, HD), lambda i: (i, 0)),
            pl.BlockSpec((1, H), lambda i: (0, 0)),
            pl.BlockSpec((1, HD), lambda i: (0, 0)),
        ],
        out_shape=[
            jax.ShapeDtypeStruct((N, SW), jnp.float32),
            jax.ShapeDtypeStruct((N, 2 * H), jnp.float32),
            jax.ShapeDtypeStruct((N, HD), jnp.float32),
            jax.ShapeDtypeStruct((1, H), jnp.float32),
            jax.ShapeDtypeStruct((1, HD), jnp.float32),
        ],
    )(x, W_node, b_node, W_src, b_src, W_rel, W_res, b_res, rel_emb2,
      W_prop, b_prop)


def _sc_body(S_hbm, D_hbm, sm_hbm, src_hbm, dst_hbm, acc_hbm,
             bufS, bufD, src_v, dst_v, sm_v, acc_s, sem):
    c = lax.axis_index("c")
    s_ = lax.axis_index("s")

    zero16 = jnp.zeros((16,), jnp.float32)
    for i in range(C):
        for j in range(SW // 16):
            bufS[i, pl.ds(16 * j, 16)] = zero16
    rb = s_ * RPT
    for k in range(7):
        pltpu.sync_copy(bufS, acc_s.at[pl.ds(rb + 80 * k, 80)])
    pltpu.sync_copy(bufS.at[pl.ds(0, RPT - 560)], acc_s.at[pl.ds(rb + 560, RPT - 560)])
    pltpu.sync_copy(sm_hbm, sm_v)
    plsc.subcore_barrier()

    base_e = c * (E // NC) + s_ * EPT

    def chunk(j, carry):
        eb = base_e + j * C
        pltpu.sync_copy(src_hbm.at[pl.ds(eb, C)], src_v)
        pltpu.sync_copy(dst_hbm.at[pl.ds(eb, C)], dst_v)
        pltpu.async_copy(S_hbm.at[src_v], bufS, sem).wait()
        pltpu.async_copy(D_hbm.at[dst_v], bufD, sem).wait()
        sm = sm_v[...]

        def edge(e, cc):
            vsrc = bufS[e, pl.ds(HD, 16)]
            vdst = bufD[e, :]
            z = vsrc + vdst
            lz = jnp.maximum(z, 0.2 * z)
            mt = jnp.maximum(vdst + sm, 0.0)
            ex = jnp.exp(lz - mt)
            bufS[e, pl.ds(HD, 16)] = ex
            for hh in range(H):
                sl = bufS[e, pl.ds(16 * hh, 16)]
                bufS[e, pl.ds(16 * hh, 16)] = sl * ex[hh]
            return cc

        lax.fori_loop(0, C, edge, 0)
        pltpu.sync_copy(bufS, acc_s.at[dst_v], add=True)
        return carry

    lax.fori_loop(0, NCH, chunk, 0)
    plsc.subcore_barrier()
    pltpu.sync_copy(acc_s.at[pl.ds(rb, RPT)], acc_hbm.at[c, pl.ds(rb, RPT)])


def _sc_edges(S, Dtab, sm16, src, dst):
    mesh = plsc.VectorSubcoreMesh(core_axis_name="c", subcore_axis_name="s")
    f = pl.kernel(
        _sc_body,
        out_type=jax.ShapeDtypeStruct((NC, N, SW), jnp.float32),
        mesh=mesh,
        compiler_params=pltpu.CompilerParams(use_tc_tiling_on_sc=False),
        scratch_types=[
            pltpu.VMEM((C, SW), jnp.float32),
            pltpu.VMEM((C, 2 * H), jnp.float32),
            pltpu.VMEM((C,), jnp.int32),
            pltpu.VMEM((C,), jnp.int32),
            pltpu.VMEM((16,), jnp.float32),
            pltpu.VMEM_SHARED((N, SW), jnp.float32),
            pltpu.SemaphoreType.DMA,
        ],
    )
    return f(S, Dtab, sm16, src, dst)


def _post_body(a0_r, a1_r, res_r, aa_r, out_r):
    num = a0_r[:, 0:HD] + a1_r[:, 0:HD]
    den = a0_r[:, HD:HD + H] + a1_r[:, HD:HD + H]
    av = aa_r[...]
    for hh in range(H):
        d1 = den[:, hh:hh + 1] + 1e-16
        agg = num[:, 16 * hh:16 * hh + 16] / d1
        out_r[:, 16 * hh:16 * hh + 16] = (
            jnp.maximum(agg, 0.0) * av
            + res_r[:, 16 * hh:16 * hh + 16] * (1.0 - av))


def _post(a0, a1, res, aa):
    BN = 1000
    return pl.pallas_call(
        _post_body,
        grid=(N // BN,),
        in_specs=[
            pl.BlockSpec((BN, SW), lambda i: (i, 0)),
            pl.BlockSpec((BN, SW), lambda i: (i, 0)),
            pl.BlockSpec((BN, HD), lambda i: (i, 0)),
            pl.BlockSpec((1, 1), lambda i: (0, 0)),
        ],
        out_specs=pl.BlockSpec((BN, HD), lambda i: (i, 0)),
        out_shape=jax.ShapeDtypeStruct((N, HD), jnp.float32),
    )(a0, a1, res, aa)


def kernel(x, edge_index, rel_emb, W_node, b_node, W_src, b_src, W_rel,
           W_res, b_res, res_w, w_cross, W_prop, b_prop):
    src = edge_index[0]
    dst = edge_index[1]
    S, Dtab, res, smax, relo = _pre(
        x, W_node, b_node.reshape(1, -1), W_src, b_src.reshape(1, -1),
        W_rel, W_res, b_res.reshape(1, -1), rel_emb.reshape(1, -1),
        W_prop, b_prop.reshape(1, -1))
    sm16 = jnp.concatenate([smax[0], smax[0]])
    acc = _sc_edges(S, Dtab, sm16, src, dst)
    aa = jnp.full((1, 1), jax.nn.sigmoid(res_w), jnp.float32)
    crossed = _post(acc[0], acc[1], res, aa)
    return crossed, relo[0]


# speedup vs baseline: 59.6231x; 1.1122x over previous
import functools

import jax
import jax.numpy as jnp
from jax import lax
from jax.experimental import pallas as pl
from jax.experimental.pallas import tpu as pltpu
from jax.experimental.pallas import tpu_sc as plsc

N = 10000
E = 320000
D_IN = 128
H = 8
D_H = 16
HD = H * D_H
SW = 144

NC = 2
NS = 16
C = 40
EPT = E // (NC * NS)
NCH = EPT // C
RPT = N // NS


def _pre_body(x_r, Wn_r, bn_r, Ws_r, bs_r, Wr_r, Wres_r, bres_r, re_r,
              Wp_r, bp_r, S_r, D_r, res_r, smax_r, relo_r):
    i = pl.program_id(0)
    xb = x_r[...]
    h = jnp.dot(xb, Wn_r[...], preferred_element_type=jnp.float32) + bn_r[...]
    s = jnp.dot(h, Ws_r[...], preferred_element_type=jnp.float32) + bs_r[...]
    S_r[:, 0:HD] = s
    res_r[...] = jnp.dot(xb, Wres_r[...], preferred_element_type=jnp.float32) + bres_r[...]
    r = jnp.dot(re_r[...], Wr_r[...], preferred_element_type=jnp.float32)
    es_l, ed_l = [], []
    for hh in range(H):
        rs = r[:, 32 * hh:32 * hh + 16]
        rd = r[:, 32 * hh + 16:32 * hh + 32]
        es_l.append(jnp.sum(s[:, 16 * hh:16 * hh + 16] * rs, axis=1, keepdims=True))
        ed_l.append(jnp.sum(h[:, 16 * hh:16 * hh + 16] * rd, axis=1, keepdims=True))
    es = jnp.concatenate(es_l, axis=1)
    ed = jnp.concatenate(ed_l, axis=1)
    S_r[:, HD:HD + H] = es
    S_r[:, HD + H:SW] = es
    D_r[:, 0:H] = ed
    D_r[:, H:2 * H] = ed

    @pl.when(i == 0)
    def _():
        smax_r[...] = jnp.full((1, H), -1e30, jnp.float32)
        relo_r[...] = jnp.dot(re_r[...], Wp_r[...],
                              preferred_element_type=jnp.float32) + bp_r[...]

    smax_r[...] = jnp.maximum(smax_r[...], jnp.max(es, axis=0, keepdims=True))


def _pre(x, W_node, b_node, W_src, b_src, W_rel, W_res, b_res, rel_emb2,
         W_prop, b_prop):
    BN = 1000
    grid = (N // BN,)
    full = lambda shp: pl.BlockSpec(shp, lambda i: (0, 0))
    return pl.pallas_call(
        _pre_body,
        grid=grid,
        in_specs=[
            pl.BlockSpec((BN, D_IN), lambda i: (i, 0)),
            full((D_IN, HD)), full((1, HD)),
            full((HD, HD)), full((1, HD)),
            full((64, 256)),
            full((D_IN, HD)), full((1, HD)),
            full((1, 64)),
            full((64, HD)), full((1, HD)),
        ],
        out_specs=[
            pl.BlockSpec((BN, SW), lambda i: (i, 0)),
            pl.BlockSpec((BN, 2 * H), lambda i: (i, 0)),
            pl.BlockSpec((BN, HD), lambda i: (i, 0)),
            pl.BlockSpec((1, H), lambda i: (0, 0)),
            pl.BlockSpec((1, HD), lambda i: (0, 0)),
        ],
        out_shape=[
            jax.ShapeDtypeStruct((N, SW), jnp.float32),
            jax.ShapeDtypeStruct((N, 2 * H), jnp.float32),
            jax.ShapeDtypeStruct((N, HD), jnp.float32),
            jax.ShapeDtypeStruct((1, H), jnp.float32),
            jax.ShapeDtypeStruct((1, HD), jnp.float32),
        ],
    )(x, W_node, b_node, W_src, b_src, W_rel, W_res, b_res, rel_emb2,
      W_prop, b_prop)


U = 8


def _sc_body(S_hbm, D_hbm, sm_hbm, src2_hbm, dst2_hbm, acc_hbm,
             bufS_a, bufS_b, bufD_a, bufD_b, src_v, dst_v, sm_v, acc_s,
             semA, semB):
    c = lax.axis_index("c")
    s_ = lax.axis_index("s")

    zero16 = jnp.zeros((16,), jnp.float32)
    for i in range(C):
        for j in range(SW // 16):
            bufS_a[i, pl.ds(16 * j, 16)] = zero16
    rb = s_ * RPT
    for k in range(RPT // C):
        pltpu.sync_copy(bufS_a, acc_s.at[pl.ds(rb + C * k, C)])
    if RPT % C:
        pltpu.sync_copy(bufS_a.at[pl.ds(0, RPT % C)],
                        acc_s.at[pl.ds(rb + (RPT // C) * C, RPT % C)])
    pltpu.sync_copy(sm_hbm, sm_v)

    tb = (c * NS + s_) * NCH
    pltpu.sync_copy(src2_hbm.at[pl.ds(tb, NCH)], src_v)
    pltpu.sync_copy(dst2_hbm.at[pl.ds(tb, NCH)], dst_v)

    def issue(j, bS, bD, sem):
        hS = pltpu.async_copy(S_hbm.at[src_v.at[j]], bS, sem)
        hD = pltpu.async_copy(D_hbm.at[dst_v.at[j]], bD, sem)
        return hS, hD

    def compute(bS, bD):
        sm = sm_v[...]

        def blk(i, cc):
            for k in range(U):
                e = i * U + k
                vsrc = bS[e, pl.ds(HD, 16)]
                vdst = bD[e, :]
                z = vsrc + vdst
                lz = jnp.maximum(z, 0.2 * z)
                mt = jnp.maximum(vdst + sm, 0.0)
                ex = jnp.exp(lz - mt)
                bS[e, pl.ds(HD, 16)] = ex
                for hh in range(H):
                    sl = bS[e, pl.ds(16 * hh, 16)]
                    bS[e, pl.ds(16 * hh, 16)] = sl * ex[hh]
            return cc

        lax.fori_loop(0, C // U, blk, 0)

    def scatter(j, bS):
        pltpu.sync_copy(bS, acc_s.at[dst_v.at[j]], add=True)

    h0S, h0D = issue(0, bufS_a, bufD_a, semA)
    h0S.wait()
    h0D.wait()
    plsc.subcore_barrier()

    def pair(t, cc):
        a = 2 * t
        hBS, hBD = issue(a + 1, bufS_b, bufD_b, semB)
        compute(bufS_a, bufD_a)
        scatter(a, bufS_a)
        hAS, hAD = issue(jnp.minimum(a + 2, NCH - 1), bufS_a, bufD_a, semA)
        hBS.wait()
        hBD.wait()
        compute(bufS_b, bufD_b)
        scatter(a + 1, bufS_b)
        hAS.wait()
        hAD.wait()
        return cc

    lax.fori_loop(0, NCH // 2, pair, 0)

    plsc.subcore_barrier()
    pltpu.sync_copy(acc_s.at[pl.ds(rb, RPT)], acc_hbm.at[c, pl.ds(rb, RPT)])


def _sc_edges(S, Dtab, sm16, src2, dst2):
    mesh = plsc.VectorSubcoreMesh(core_axis_name="c", subcore_axis_name="s")
    f = pl.kernel(
        _sc_body,
        out_type=jax.ShapeDtypeStruct((NC, N, SW), jnp.float32),
        mesh=mesh,
        compiler_params=pltpu.CompilerParams(use_tc_tiling_on_sc=False),
        scratch_types=[
            pltpu.VMEM((C, SW), jnp.float32),
            pltpu.VMEM((C, SW), jnp.float32),
            pltpu.VMEM((C, 2 * H), jnp.float32),
            pltpu.VMEM((C, 2 * H), jnp.float32),
            pltpu.VMEM((NCH, C), jnp.int32),
            pltpu.VMEM((NCH, C), jnp.int32),
            pltpu.VMEM((16,), jnp.float32),
            pltpu.VMEM_SHARED((N, SW), jnp.float32),
            pltpu.SemaphoreType.DMA,
            pltpu.SemaphoreType.DMA,
        ],
    )
    return f(S, Dtab, sm16, src2, dst2)


def _post_body(a0_r, a1_r, res_r, aa_r, out_r):
    num = a0_r[:, 0:HD] + a1_r[:, 0:HD]
    den = a0_r[:, HD:HD + H] + a1_r[:, HD:HD + H]
    av = aa_r[...]
    for hh in range(H):
        d1 = den[:, hh:hh + 1] + 1e-16
        agg = num[:, 16 * hh:16 * hh + 16] / d1
        out_r[:, 16 * hh:16 * hh + 16] = (
            jnp.maximum(agg, 0.0) * av
            + res_r[:, 16 * hh:16 * hh + 16] * (1.0 - av))


def _post(a0, a1, res, aa):
    BN = 1000
    return pl.pallas_call(
        _post_body,
        grid=(N // BN,),
        in_specs=[
            pl.BlockSpec((BN, SW), lambda i: (i, 0)),
            pl.BlockSpec((BN, SW), lambda i: (i, 0)),
            pl.BlockSpec((BN, HD), lambda i: (i, 0)),
            pl.BlockSpec((1, 1), lambda i: (0, 0)),
        ],
        out_specs=pl.BlockSpec((BN, HD), lambda i: (i, 0)),
        out_shape=jax.ShapeDtypeStruct((N, HD), jnp.float32),
    )(a0, a1, res, aa)


def kernel(x, edge_index, rel_emb, W_node, b_node, W_src, b_src, W_rel,
           W_res, b_res, res_w, w_cross, W_prop, b_prop):
    src = edge_index[0].reshape(NC * NS * NCH, C)
    dst = edge_index[1].reshape(NC * NS * NCH, C)
    S, Dtab, res, smax, relo = _pre(
        x, W_node, b_node.reshape(1, -1), W_src, b_src.reshape(1, -1),
        W_rel, W_res, b_res.reshape(1, -1), rel_emb.reshape(1, -1),
        W_prop, b_prop.reshape(1, -1))
    sm16 = jnp.concatenate([smax[0], smax[0]])
    acc = _sc_edges(S, Dtab, sm16, src, dst)
    aa = jnp.full((1, 1), jax.nn.sigmoid(res_w), jnp.float32)
    crossed = _post(acc[0], acc[1], res, aa)
    return crossed, relo[0]


# speedup vs baseline: 111.8701x; 1.8763x over previous
import functools

import jax
import jax.numpy as jnp
import numpy as np
from jax import lax
from jax.experimental import pallas as pl
from jax.experimental.pallas import tpu as pltpu
from jax.experimental.pallas import tpu_sc as plsc

N = 10000
E = 320000
D_IN = 128
R_IN_ = 64
H = 8
D_H = 16
HD = H * D_H
SW = 144

NC = 2
NS = 16
C = 80
EPT = E // (NC * NS)
NCH = EPT // C
RPT = N // NS

_PCOL = np.array([16 * (j % 8) + 2 * (j // 16) + (j % 16) // 8
                  for j in range(HD)], dtype=np.int32)
_PMAT = np.zeros((HD, HD), np.float32)
_PMAT[np.arange(HD), _PCOL] = 1.0


def _pre_body(x_r, Wn_r, bn_r, Ws_r, bs_r, Wr_r, Wres_r, bres_r, re_r,
              Wp_r, bp_r, ei_r, S_r, D_r, res_r, smax_r, relo_r, pk_r):
    i = pl.program_id(0)
    pk_r[...] = (ei_r[0:1, :] + ei_r[1:2, :] * 65536).reshape(pk_r.shape)
    xb = x_r[...]
    hp = lambda a, b: jnp.dot(a, b, preferred_element_type=jnp.float32,
                              precision=lax.Precision.HIGHEST)
    cc2 = lax.broadcasted_iota(jnp.int32, (HD, HD), 0)
    jj2 = lax.broadcasted_iota(jnp.int32, (HD, HD), 1)
    pmt = (cc2 == 16 * (jj2 % H) + 2 * (jj2 // D_H)
           + (jj2 % D_H) // H).astype(jnp.float32)
    h = hp(xb, Wn_r[...]) + bn_r[...]
    s = hp(hp(h, Ws_r[...]) + bs_r[...], pmt)
    S_r[:, 0:HD] = s
    res_r[...] = hp(hp(xb, Wres_r[...]) + bres_r[...], pmt)
    r = hp(re_r[...], Wr_r[...])
    rs = jnp.concatenate([r[:, 32 * k:32 * k + 16] for k in range(H)], axis=1)
    rd = jnp.concatenate([r[:, 32 * k + 16:32 * k + 32] for k in range(H)], axis=1)
    rsp = hp(rs, pmt)
    iota_j = lax.broadcasted_iota(jnp.int32, (HD, H), 0)
    iota_h = lax.broadcasted_iota(jnp.int32, (HD, H), 1)
    m8 = (iota_j % H == iota_h).astype(jnp.float32)
    m16 = (iota_j // D_H == iota_h).astype(jnp.float32)
    es = hp(s * rsp, m8)
    ed = hp(h * rd, m16)
    S_r[:, HD:HD + H] = es
    S_r[:, HD + H:SW] = es
    D_r[:, 0:H] = ed
    D_r[:, H:2 * H] = ed

    @pl.when(i == 0)
    def _():
        smax_r[...] = jnp.full((1, 2 * H), -1e30, jnp.float32)
        relo_r[...] = hp(re_r[...], Wp_r[...]) + bp_r[...]

    bm = jnp.max(es, axis=0, keepdims=True)
    smax_r[...] = jnp.maximum(smax_r[...],
                              jnp.concatenate([bm, bm], axis=1))


def _pre(x, W_node, b_node, W_src, b_src, W_rel, W_res, b_res, rel_emb2,
         W_prop, b_prop, edge_index):
    BN = 1000
    grid = (N // BN,)
    BE = E // (N // BN)
    full = lambda shp: pl.BlockSpec(shp, lambda i: (0, 0))
    return pl.pallas_call(
        _pre_body,
        grid=grid,
        in_specs=[
            pl.BlockSpec((BN, D_IN), lambda i: (i, 0)),
            full((D_IN, HD)), full((1, HD)),
            full((HD, HD)), full((1, HD)),
            full((64, 256)),
            full((D_IN, HD)), full((1, HD)),
            full((1, 64)),
            full((64, HD)), full((1, HD)),
            pl.BlockSpec((2, BE), lambda i: (0, i)),
        ],
        out_specs=[
            pl.BlockSpec((BN, SW), lambda i: (i, 0)),
            pl.BlockSpec((BN, 2 * H), lambda i: (i, 0)),
            pl.BlockSpec((BN, HD), lambda i: (i, 0)),
            pl.BlockSpec((1, 2 * H), lambda i: (0, 0)),
            pl.BlockSpec((1, HD), lambda i: (0, 0)),
            pl.BlockSpec((1, 1, BE), lambda i: (i, 0, 0)),
        ],
        out_shape=[
            jax.ShapeDtypeStruct((N, SW), jnp.float32),
            jax.ShapeDtypeStruct((N, 2 * H), jnp.float32),
            jax.ShapeDtypeStruct((N, HD), jnp.float32),
            jax.ShapeDtypeStruct((1, 2 * H), jnp.float32),
            jax.ShapeDtypeStruct((1, HD), jnp.float32),
            jax.ShapeDtypeStruct((N // BN, 1, BE), jnp.int32),
        ],
    )(x, W_node, b_node, W_src, b_src, W_rel, W_res, b_res, rel_emb2,
      W_prop, b_prop, edge_index)


U = 8


def _sc_body(S_hbm, D_hbm, sm_hbm, pk2_hbm, acc_hbm,
             bufS_a, bufS_b, bufD_a, bufD_b, pk_v,
             srcI_a, srcI_b, dstI_a, dstI_b, sm_v, acc_s,
             semA, semB, semSA, semSB):
    c = lax.axis_index("c")
    s_ = lax.axis_index("s")

    zero16 = jnp.zeros((16,), jnp.float32)
    for i in range(C):
        for j in range(SW // 16):
            bufS_a[i, pl.ds(16 * j, 16)] = zero16
    rb = s_ * RPT
    for k in range(RPT // C):
        pltpu.sync_copy(bufS_a, acc_s.at[pl.ds(rb + C * k, C)])
    if RPT % C:
        pltpu.sync_copy(bufS_a.at[pl.ds(0, RPT % C)],
                        acc_s.at[pl.ds(rb + (RPT // C) * C, RPT % C)])
    pltpu.sync_copy(sm_hbm, sm_v)

    tb = (c * NS + s_) * NCH
    pltpu.sync_copy(pk2_hbm.at[pl.ds(tb, NCH)], pk_v)

    def unpack(j, sI, dI):
        for g in range(C // 16):
            pv = pk_v[j, pl.ds(16 * g, 16)]
            sI[pl.ds(16 * g, 16)] = lax.bitwise_and(pv, 0xFFFF)
            dI[pl.ds(16 * g, 16)] = lax.shift_right_logical(pv, 16)

    def issue(bS, bD, sI, dI, sem):
        hS = pltpu.async_copy(S_hbm.at[sI], bS, sem)
        hD = pltpu.async_copy(D_hbm.at[dI], bD, sem)
        return hS, hD

    def compute(bS, bD):
        sm = sm_v[...]

        @plsc.parallel_loop(0, C, unroll=U)
        def _(e):
            vsrc = bS[e, pl.ds(HD, 16)]
            vdst = bD[e, :]
            z = vsrc + vdst
            lz = jnp.maximum(z, 0.2 * z)
            zb = vdst + sm
            mt = jnp.maximum(zb, 0.2 * zb)
            ex = jnp.exp(lz - mt)
            bS[e, pl.ds(HD, 16)] = ex
            for k in range(H):
                sl = bS[e, pl.ds(16 * k, 16)]
                bS[e, pl.ds(16 * k, 16)] = sl * ex

    def scatter(bS, dI, sem):
        return pltpu.async_copy(bS, acc_s.at[dI], sem, add=True)

    unpack(0, srcI_a, dstI_a)
    h0S, h0D = issue(bufS_a, bufD_a, srcI_a, dstI_a, semA)
    h0S.wait()
    h0D.wait()
    plsc.subcore_barrier()

    def pair(t, cc):
        a = 2 * t
        unpack(a + 1, srcI_b, dstI_b)
        hBS, hBD = issue(bufS_b, bufD_b, srcI_b, dstI_b, semB)
        compute(bufS_a, bufD_a)
        hSA = scatter(bufS_a, dstI_a, semSA)
        hBS.wait()
        hBD.wait()
        compute(bufS_b, bufD_b)
        hSB = scatter(bufS_b, dstI_b, semSB)
        hSA.wait()
        unpack(a + 2, srcI_a, dstI_a)
        hAS, hAD = issue(bufS_a, bufD_a, srcI_a, dstI_a, semA)
        hSB.wait()
        hAS.wait()
        hAD.wait()
        return cc

    lax.fori_loop(0, (NCH - 1) // 2, pair, 0)
    compute(bufS_a, bufD_a)
    scatter(bufS_a, dstI_a, semSA).wait()

    plsc.subcore_barrier()
    pltpu.sync_copy(acc_s.at[pl.ds(rb, RPT)], acc_hbm.at[c, pl.ds(rb, RPT)])


def _sc_edges(S, Dtab, sm16, pk2):
    mesh = plsc.VectorSubcoreMesh(core_axis_name="c", subcore_axis_name="s")
    f = pl.kernel(
        _sc_body,
        out_type=jax.ShapeDtypeStruct((NC, N, SW), jnp.float32),
        mesh=mesh,
        compiler_params=pltpu.CompilerParams(use_tc_tiling_on_sc=False),
        scratch_types=[
            pltpu.VMEM((C, SW), jnp.float32),
            pltpu.VMEM((C, SW), jnp.float32),
            pltpu.VMEM((C, 2 * H), jnp.float32),
            pltpu.VMEM((C, 2 * H), jnp.float32),
            pltpu.VMEM((NCH, C), jnp.int32),
            pltpu.VMEM((C,), jnp.int32),
            pltpu.VMEM((C,), jnp.int32),
            pltpu.VMEM((C,), jnp.int32),
            pltpu.VMEM((C,), jnp.int32),
            pltpu.VMEM((16,), jnp.float32),
            pltpu.VMEM_SHARED((N, SW), jnp.float32),
            pltpu.SemaphoreType.DMA,
            pltpu.SemaphoreType.DMA,
            pltpu.SemaphoreType.DMA,
            pltpu.SemaphoreType.DMA,
        ],
    )
    return f(S, Dtab, sm16, pk2)


def _post_body(acc_r, res_r, aa_r, out_r):
    a0_r = acc_r[0]
    a1_r = acc_r[1]
    num = a0_r[:, 0:HD] + a1_r[:, 0:HD]
    den = a0_r[:, HD:HD + H] + a1_r[:, HD:HD + H]
    den16 = jnp.maximum(jnp.concatenate([den, den], axis=1), 1e-30)
    av = 1.0 / (1.0 + jnp.exp(-aa_r[...]))
    cols = []
    for k in range(H):
        agg = num[:, 16 * k:16 * k + 16] / den16
        cols.append(jnp.maximum(agg, 0.0) * av
                    + res_r[:, 16 * k:16 * k + 16] * (1.0 - av))
    outp = jnp.concatenate(cols, axis=1)
    jj = lax.broadcasted_iota(jnp.int32, (HD, HD), 0)
    cc = lax.broadcasted_iota(jnp.int32, (HD, HD), 1)
    pm = (16 * (jj % H) + 2 * (jj // D_H) + (jj % D_H) // H == cc)
    out_r[...] = jnp.dot(outp, pm.astype(jnp.float32),
                         preferred_element_type=jnp.float32, precision=lax.Precision.HIGHEST)


def _post(acc, res, aa):
    BN = 1000
    return pl.pallas_call(
        _post_body,
        grid=(N // BN,),
        in_specs=[
            pl.BlockSpec((NC, BN, SW), lambda i: (0, i, 0)),
            pl.BlockSpec((BN, HD), lambda i: (i, 0)),
            pl.BlockSpec((1, 1), lambda i: (0, 0)),
        ],
        out_specs=pl.BlockSpec((BN, HD), lambda i: (i, 0)),
        out_shape=jax.ShapeDtypeStruct((N, HD), jnp.float32),
    )(acc, res, aa)


def kernel(x, edge_index, rel_emb, W_node, b_node, W_src, b_src, W_rel,
           W_res, b_res, res_w, w_cross, W_prop, b_prop):
    S, Dtab, res, smax, relo, pk = _pre(
        x, W_node, b_node.reshape(1, -1), W_src, b_src.reshape(1, -1),
        W_rel, W_res, b_res.reshape(1, -1), rel_emb.reshape(1, -1),
        W_prop, b_prop.reshape(1, -1), edge_index)
    acc = _sc_edges(S, Dtab, smax[0], pk.reshape(NC * NS * NCH, C))
    crossed = _post(acc, res, res_w.reshape(1, 1))
    return crossed, relo[0]


# speedup vs baseline: 124.3990x; 1.1120x over previous
import functools

import jax
import jax.numpy as jnp
import numpy as np
from jax import lax
from jax.experimental import pallas as pl
from jax.experimental.pallas import tpu as pltpu
from jax.experimental.pallas import tpu_sc as plsc

N = 10000
E = 320000
D_IN = 128
R_IN_ = 64
H = 8
D_H = 16
HD = H * D_H
SW = 144

NC = 2
NS = 16
C = 40
EPT = E // (NC * NS)
NCH = EPT // C
RPT = N // NS

_PCOL = np.array([16 * (j % 8) + 2 * (j // 16) + (j % 16) // 8
                  for j in range(HD)], dtype=np.int32)
_PMAT = np.zeros((HD, HD), np.float32)
_PMAT[np.arange(HD), _PCOL] = 1.0


def _pre_body(x_r, Wn_r, bn_r, Ws_r, bs_r, Wr_r, Wres_r, bres_r, re_r,
              Wp_r, bp_r, S_r, D_r, res_r, smax_r, relo_r):
    i = pl.program_id(0)
    xb = x_r[...]
    hp = lambda a, b: jnp.dot(a, b, preferred_element_type=jnp.float32,
                              precision=lax.Precision.HIGHEST)
    dd = lambda a, b: jnp.dot(a, b, preferred_element_type=jnp.float32)
    h = dd(xb, Wn_r[...]) + bn_r[...]
    s = dd(h, Ws_r[...]) + bs_r[...]
    S_r[:, 0:HD] = s
    res_r[...] = dd(xb, Wres_r[...]) + bres_r[...]
    rr = hp(re_r[...], Wr_r[...])
    iota_j = lax.broadcasted_iota(jnp.int32, (HD, H), 0)
    iota_h = lax.broadcasted_iota(jnp.int32, (HD, H), 1)
    m8 = (iota_j % H == iota_h).astype(jnp.float32)
    m16 = (iota_j // D_H == iota_h).astype(jnp.float32)
    es = hp(s * rr[:, 0:HD], m8)
    ed = hp(h * rr[:, HD:2 * HD], m16)
    S_r[:, HD:HD + H] = es
    S_r[:, HD + H:SW] = es
    D_r[:, 0:H] = ed
    D_r[:, H:2 * H] = ed

    @pl.when(i == 0)
    def _():
        smax_r[...] = jnp.full((1, 2 * H), -1e30, jnp.float32)
        relo_r[...] = hp(re_r[...], Wp_r[...]) + bp_r[...]

    bm = jnp.max(es, axis=0, keepdims=True)
    smax_r[...] = jnp.maximum(smax_r[...],
                              jnp.concatenate([bm, bm], axis=1))


def _pre(x, W_node, b_node, W_src, b_src, W_rel, W_res, b_res, rel_emb2,
         W_prop, b_prop):
    BN = 1000
    grid = (N // BN,)
    full = lambda shp: pl.BlockSpec(shp, lambda i: (0, 0))
    return pl.pallas_call(
        _pre_body,
        grid=grid,
        in_specs=[
            pl.BlockSpec((BN, D_IN), lambda i: (i, 0)),
            full((D_IN, HD)), full((1, HD)),
            full((HD, HD)), full((1, HD)),
            full((64, 256)),
            full((D_IN, HD)), full((1, HD)),
            full((1, 64)),
            full((64, HD)), full((1, HD)),
        ],
        out_specs=[
            pl.BlockSpec((BN, SW), lambda i: (i, 0)),
            pl.BlockSpec((BN, 2 * H), lambda i: (i, 0)),
            pl.BlockSpec((BN, HD), lambda i: (i, 0)),
            pl.BlockSpec((1, 2 * H), lambda i: (0, 0)),
            pl.BlockSpec((1, HD), lambda i: (0, 0)),
        ],
        out_shape=[
            jax.ShapeDtypeStruct((N, SW), jnp.float32),
            jax.ShapeDtypeStruct((N, 2 * H), jnp.float32),
            jax.ShapeDtypeStruct((N, HD), jnp.float32),
            jax.ShapeDtypeStruct((1, 2 * H), jnp.float32),
            jax.ShapeDtypeStruct((1, HD), jnp.float32),
        ],
    )(x, W_node, b_node, W_src, b_src, W_rel, W_res, b_res, rel_emb2,
      W_prop, b_prop)


U = 8


def _sc_body(S_hbm, D_hbm, sm_hbm, src2_hbm, dst2_hbm, acc_hbm,
             bufS_a, bufS_b, bufD_a, bufD_b, src_v, dst_v, sm_v, acc_s,
             semA, semB):
    c = lax.axis_index("c")
    s_ = lax.axis_index("s")

    zero16 = jnp.zeros((16,), jnp.float32)
    for i in range(C):
        for j in range(SW // 16):
            bufS_a[i, pl.ds(16 * j, 16)] = zero16
    rb = s_ * RPT
    for k in range(RPT // C):
        pltpu.sync_copy(bufS_a, acc_s.at[pl.ds(rb + C * k, C)])
    if RPT % C:
        pltpu.sync_copy(bufS_a.at[pl.ds(0, RPT % C)],
                        acc_s.at[pl.ds(rb + (RPT // C) * C, RPT % C)])
    pltpu.sync_copy(sm_hbm, sm_v)

    tb = (c * NS + s_) * NCH
    pltpu.sync_copy(src2_hbm.at[pl.ds(tb, NCH)], src_v)
    pltpu.sync_copy(dst2_hbm.at[pl.ds(tb, NCH)], dst_v)

    def issue(j, bS, bD, sem):
        hS = pltpu.async_copy(S_hbm.at[src_v.at[j]], bS, sem)
        hD = pltpu.async_copy(D_hbm.at[dst_v.at[j]], bD, sem)
        return hS, hD

    def compute(bS, bD):
        sm = sm_v[...]

        @plsc.parallel_loop(0, C, unroll=U)
        def _(e):
            vsrc = bS[e, pl.ds(HD, 16)]
            vdst = bD[e, :]
            z = vsrc + vdst
            lz = jnp.maximum(z, 0.2 * z)
            zb = vdst + sm
            mt = jnp.maximum(zb, 0.2 * zb)
            ex = jnp.exp(lz - mt)
            bS[e, pl.ds(HD, 16)] = ex
            for k in range(H):
                sl = bS[e, pl.ds(16 * k, 16)]
                bS[e, pl.ds(16 * k, 16)] = sl * ex

    def scatter(j, bS):
        pltpu.sync_copy(bS, acc_s.at[dst_v.at[j]], add=True)

    h0S, h0D = issue(0, bufS_a, bufD_a, semA)
    h0S.wait()
    h0D.wait()
    plsc.subcore_barrier()

    def pair(t, cc):
        a = 2 * t
        hBS, hBD = issue(a + 1, bufS_b, bufD_b, semB)
        compute(bufS_a, bufD_a)
        scatter(a, bufS_a)
        hAS, hAD = issue(jnp.minimum(a + 2, NCH - 1), bufS_a, bufD_a, semA)
        hBS.wait()
        hBD.wait()
        compute(bufS_b, bufD_b)
        scatter(a + 1, bufS_b)
        hAS.wait()
        hAD.wait()
        return cc

    lax.fori_loop(0, NCH // 2, pair, 0)

    plsc.subcore_barrier()
    pltpu.sync_copy(acc_s.at[pl.ds(rb, RPT)], acc_hbm.at[c, pl.ds(rb, RPT)])


def _sc_edges(S, Dtab, sm16, src2, dst2):
    mesh = plsc.VectorSubcoreMesh(core_axis_name="c", subcore_axis_name="s")
    f = pl.kernel(
        _sc_body,
        out_type=jax.ShapeDtypeStruct((NC, N, SW), jnp.float32),
        mesh=mesh,
        compiler_params=pltpu.CompilerParams(use_tc_tiling_on_sc=False),
        scratch_types=[
            pltpu.VMEM((C, SW), jnp.float32),
            pltpu.VMEM((C, SW), jnp.float32),
            pltpu.VMEM((C, 2 * H), jnp.float32),
            pltpu.VMEM((C, 2 * H), jnp.float32),
            pltpu.VMEM((NCH, C), jnp.int32),
            pltpu.VMEM((NCH, C), jnp.int32),
            pltpu.VMEM((16,), jnp.float32),
            pltpu.VMEM_SHARED((N, SW), jnp.float32),
            pltpu.SemaphoreType.DMA,
            pltpu.SemaphoreType.DMA,
        ],
    )
    return f(S, Dtab, sm16, src2, dst2)


def _post_body(acc_r, res_r, aa_r, out_r):
    a0_r = acc_r[0]
    a1_r = acc_r[1]
    num = a0_r[:, 0:HD] + a1_r[:, 0:HD]
    den = a0_r[:, HD:HD + H] + a1_r[:, HD:HD + H]
    den16 = jnp.maximum(jnp.concatenate([den, den], axis=1), 1e-30)
    av = 1.0 / (1.0 + jnp.exp(-aa_r[...]))
    cols = []
    for k in range(H):
        agg = num[:, 16 * k:16 * k + 16] / den16
        cols.append(jnp.maximum(agg, 0.0) * av
                    + res_r[:, 16 * k:16 * k + 16] * (1.0 - av))
    outp = jnp.concatenate(cols, axis=1)
    jj = lax.broadcasted_iota(jnp.int32, (HD, HD), 0)
    cc = lax.broadcasted_iota(jnp.int32, (HD, HD), 1)
    pm = (16 * (jj % H) + 2 * (jj // D_H) + (jj % D_H) // H == cc)
    out_r[...] = jnp.dot(outp, pm.astype(jnp.float32),
                         preferred_element_type=jnp.float32, precision=lax.Precision.HIGHEST)


def _post(acc, res, aa):
    BN = 1000
    return pl.pallas_call(
        _post_body,
        grid=(N // BN,),
        in_specs=[
            pl.BlockSpec((NC, BN, SW), lambda i: (0, i, 0)),
            pl.BlockSpec((BN, HD), lambda i: (i, 0)),
            pl.BlockSpec((1, 1), lambda i: (0, 0)),
        ],
        out_specs=pl.BlockSpec((BN, HD), lambda i: (i, 0)),
        out_shape=jax.ShapeDtypeStruct((N, HD), jnp.float32),
    )(acc, res, aa)


def kernel(x, edge_index, rel_emb, W_node, b_node, W_src, b_src, W_rel,
           W_res, b_res, res_w, w_cross, W_prop, b_prop):
    pcol = jnp.asarray(_PCOL)
    wrel4 = W_rel.reshape(R_IN_, H, 2, D_H)
    wrel_re = jnp.concatenate(
        [wrel4[:, :, 0, :].reshape(R_IN_, HD)[:, pcol],
         wrel4[:, :, 1, :].reshape(R_IN_, HD)], axis=1)
    S, Dtab, res, smax, relo = _pre(
        x, W_node, b_node.reshape(1, -1), W_src[:, pcol],
        b_src[pcol].reshape(1, -1), wrel_re, W_res[:, pcol],
        b_res[pcol].reshape(1, -1), rel_emb.reshape(1, -1),
        W_prop, b_prop.reshape(1, -1))
    acc = _sc_edges(S, Dtab, smax[0],
                    edge_index[0].reshape(NC * NS * NCH, C),
                    edge_index[1].reshape(NC * NS * NCH, C))
    crossed = _post(acc, res, res_w.reshape(1, 1))
    return crossed, relo[0]


# speedup vs baseline: 128.6533x; 1.0342x over previous
import jax
import jax.numpy as jnp
import numpy as np
from jax import lax
from jax.experimental import pallas as pl
from jax.experimental.pallas import tpu as pltpu
from jax.experimental.pallas import tpu_sc as plsc

N = 10000
E = 320000
D_IN = 128
R_IN_ = 64
H = 8
D_H = 16
HD = H * D_H
SW = 144

NC = 2
NS = 16
C = 40
EPT = E // (NC * NS)
NCH = EPT // C
RPT = N // NS

_PCOL = np.array([16 * (j % 8) + 2 * (j // 16) + (j % 16) // 8
                  for j in range(HD)], dtype=np.int32)
_PMAT = np.zeros((HD, HD), np.float32)
_PMAT[np.arange(HD), _PCOL] = 1.0


def _pre_body(x_r, Wn_r, bn_r, Ws_r, bs_r, Wr_r, Wres_r, bres_r, re_r,
              Wp_r, bp_r, S_r, D_r, res_r, smax_r, relo_r):
    i = pl.program_id(0)
    xb = x_r[...]
    hp = lambda a, b: jnp.dot(a, b, preferred_element_type=jnp.float32,
                              precision=lax.Precision.HIGHEST)
    dd = lambda a, b: jnp.dot(a, b, preferred_element_type=jnp.float32)
    h = dd(xb, Wn_r[...]) + bn_r[...]
    s = dd(h, Ws_r[...]) + bs_r[...]
    S_r[:, 0:HD] = s
    res_r[...] = dd(xb, Wres_r[...]) + bres_r[...]
    rr = hp(re_r[...], Wr_r[...])
    iota_j = lax.broadcasted_iota(jnp.int32, (HD, H), 0)
    iota_h = lax.broadcasted_iota(jnp.int32, (HD, H), 1)
    m8 = (iota_j % H == iota_h).astype(jnp.float32)
    m16 = (iota_j // D_H == iota_h).astype(jnp.float32)
    es = hp(s * rr[:, 0:HD], m8)
    ed = hp(h * rr[:, HD:2 * HD], m16)
    S_r[:, HD:HD + H] = es
    S_r[:, HD + H:SW] = es
    D_r[:, 0:H] = ed
    D_r[:, H:2 * H] = ed

    @pl.when(i == 0)
    def _():
        smax_r[...] = jnp.full((1, 2 * H), -1e30, jnp.float32)
        relo_r[...] = hp(re_r[...], Wp_r[...]) + bp_r[...]

    bm = jnp.max(es, axis=0, keepdims=True)
    smax_r[...] = jnp.maximum(smax_r[...],
                              jnp.concatenate([bm, bm], axis=1))


def _pre(x, W_node, b_node, W_src, b_src, W_rel, W_res, b_res, rel_emb2,
         W_prop, b_prop):
    BN = 2000
    grid = (N // BN,)
    full = lambda shp: pl.BlockSpec(shp, lambda i: (0, 0))
    return pl.pallas_call(
        _pre_body,
        grid=grid,
        in_specs=[
            pl.BlockSpec((BN, D_IN), lambda i: (i, 0)),
            full((D_IN, HD)), full((1, HD)),
            full((HD, HD)), full((1, HD)),
            full((64, 256)),
            full((D_IN, HD)), full((1, HD)),
            full((1, 64)),
            full((64, HD)), full((1, HD)),
        ],
        out_specs=[
            pl.BlockSpec((BN, SW), lambda i: (i, 0)),
            pl.BlockSpec((BN, 2 * H), lambda i: (i, 0)),
            pl.BlockSpec((BN, HD), lambda i: (i, 0)),
            pl.BlockSpec((1, 2 * H), lambda i: (0, 0)),
            pl.BlockSpec((1, HD), lambda i: (0, 0)),
        ],
        out_shape=[
            jax.ShapeDtypeStruct((N, SW), jnp.float32),
            jax.ShapeDtypeStruct((N, 2 * H), jnp.float32),
            jax.ShapeDtypeStruct((N, HD), jnp.float32),
            jax.ShapeDtypeStruct((1, 2 * H), jnp.float32),
            jax.ShapeDtypeStruct((1, HD), jnp.float32),
        ],
    )(x, W_node, b_node, W_src, b_src, W_rel, W_res, b_res, rel_emb2,
      W_prop, b_prop)


U = 8


def _sc_body(S_hbm, D_hbm, sm_hbm, src2_hbm, dst2_hbm, acc_hbm,
             bufS_a, bufS_b, bufD_a, bufD_b, src_v, dst_v, sm_v, acc_s,
             semA, semB):
    c = lax.axis_index("c")
    s_ = lax.axis_index("s")

    zero16 = jnp.zeros((16,), jnp.float32)
    for i in range(C):
        for j in range(SW // 16):
            bufS_a[i, pl.ds(16 * j, 16)] = zero16
    rb = s_ * RPT
    for k in range(RPT // C):
        pltpu.sync_copy(bufS_a, acc_s.at[pl.ds(rb + C * k, C)])
    if RPT % C:
        pltpu.sync_copy(bufS_a.at[pl.ds(0, RPT % C)],
                        acc_s.at[pl.ds(rb + (RPT // C) * C, RPT % C)])
    pltpu.sync_copy(sm_hbm, sm_v)

    tb = (c * NS + s_) * NCH
    pltpu.sync_copy(src2_hbm.at[pl.ds(tb, NCH)], src_v)
    pltpu.sync_copy(dst2_hbm.at[pl.ds(tb, NCH)], dst_v)

    def issue(j, bS, bD, sem):
        hS = pltpu.async_copy(S_hbm.at[src_v.at[j]], bS, sem)
        hD = pltpu.async_copy(D_hbm.at[dst_v.at[j]], bD, sem)
        return hS, hD

    def compute(bS, bD):
        sm = sm_v[...]

        @plsc.parallel_loop(0, C, unroll=U)
        def _(e):
            vsrc = bS[e, pl.ds(HD, 16)]
            vdst = bD[e, :]
            z = vsrc + vdst
            lz = jnp.maximum(z, 0.2 * z)
            zb = vdst + sm
            mt = jnp.maximum(zb, 0.2 * zb)
            ex = jnp.exp(lz - mt)
            bS[e, pl.ds(HD, 16)] = ex
            for k in range(H):
                sl = bS[e, pl.ds(16 * k, 16)]
                bS[e, pl.ds(16 * k, 16)] = sl * ex

    def scatter(j, bS):
        pltpu.sync_copy(bS, acc_s.at[dst_v.at[j]], add=True)

    h0S, h0D = issue(0, bufS_a, bufD_a, semA)
    h0S.wait()
    h0D.wait()
    plsc.subcore_barrier()

    def pair(t, cc):
        a = 2 * t
        hBS, hBD = issue(a + 1, bufS_b, bufD_b, semB)
        compute(bufS_a, bufD_a)
        scatter(a, bufS_a)
        hAS, hAD = issue(jnp.minimum(a + 2, NCH - 1), bufS_a, bufD_a, semA)
        hBS.wait()
        hBD.wait()
        compute(bufS_b, bufD_b)
        scatter(a + 1, bufS_b)
        hAS.wait()
        hAD.wait()
        return cc

    lax.fori_loop(0, NCH // 2, pair, 0)

    plsc.subcore_barrier()
    pltpu.sync_copy(acc_s.at[pl.ds(rb, RPT)], acc_hbm.at[c, pl.ds(rb, RPT)])


def _sc_edges(S, Dtab, sm16, src2, dst2):
    mesh = plsc.VectorSubcoreMesh(core_axis_name="c", subcore_axis_name="s")
    f = pl.kernel(
        _sc_body,
        out_type=jax.ShapeDtypeStruct((NC, N, SW), jnp.float32),
        mesh=mesh,
        compiler_params=pltpu.CompilerParams(use_tc_tiling_on_sc=False),
        scratch_types=[
            pltpu.VMEM((C, SW), jnp.float32),
            pltpu.VMEM((C, SW), jnp.float32),
            pltpu.VMEM((C, 2 * H), jnp.float32),
            pltpu.VMEM((C, 2 * H), jnp.float32),
            pltpu.VMEM((NCH, C), jnp.int32),
            pltpu.VMEM((NCH, C), jnp.int32),
            pltpu.VMEM((16,), jnp.float32),
            pltpu.VMEM_SHARED((N, SW), jnp.float32),
            pltpu.SemaphoreType.DMA,
            pltpu.SemaphoreType.DMA,
        ],
    )
    return f(S, Dtab, sm16, src2, dst2)


def _post_body(acc_r, res_r, aa_r, out_r):
    a0_r = acc_r[0]
    a1_r = acc_r[1]
    num = a0_r[:, 0:HD] + a1_r[:, 0:HD]
    den = a0_r[:, HD:HD + H] + a1_r[:, HD:HD + H]
    den16 = jnp.maximum(jnp.concatenate([den, den], axis=1), 1e-30)
    av = 1.0 / (1.0 + jnp.exp(-aa_r[...]))
    cols = []
    for k in range(H):
        agg = num[:, 16 * k:16 * k + 16] / den16
        cols.append(jnp.maximum(agg, 0.0) * av
                    + res_r[:, 16 * k:16 * k + 16] * (1.0 - av))
    outp = jnp.concatenate(cols, axis=1)
    jj = lax.broadcasted_iota(jnp.int32, (HD, HD), 0)
    cc = lax.broadcasted_iota(jnp.int32, (HD, HD), 1)
    pm = (16 * (jj % H) + 2 * (jj // D_H) + (jj % D_H) // H == cc)
    out_r[...] = jnp.dot(outp, pm.astype(jnp.float32),
                         preferred_element_type=jnp.float32, precision=lax.Precision.HIGHEST)


def _post(acc, res, aa):
    BN = 2000
    return pl.pallas_call(
        _post_body,
        grid=(N // BN,),
        in_specs=[
            pl.BlockSpec((NC, BN, SW), lambda i: (0, i, 0)),
            pl.BlockSpec((BN, HD), lambda i: (i, 0)),
            pl.BlockSpec((1, 1), lambda i: (0, 0)),
        ],
        out_specs=pl.BlockSpec((BN, HD), lambda i: (i, 0)),
        out_shape=jax.ShapeDtypeStruct((N, HD), jnp.float32),
    )(acc, res, aa)


def kernel(x, edge_index, rel_emb, W_node, b_node, W_src, b_src, W_rel,
           W_res, b_res, res_w, w_cross, W_prop, b_prop):
    pcol = jnp.asarray(_PCOL)
    wrel4 = W_rel.reshape(R_IN_, H, 2, D_H)
    wrel_re = jnp.concatenate(
        [wrel4[:, :, 0, :].reshape(R_IN_, HD)[:, pcol],
         wrel4[:, :, 1, :].reshape(R_IN_, HD)], axis=1)
    S, Dtab, res, smax, relo = _pre(
        x, W_node, b_node.reshape(1, -1), W_src[:, pcol],
        b_src[pcol].reshape(1, -1), wrel_re, W_res[:, pcol],
        b_res[pcol].reshape(1, -1), rel_emb.reshape(1, -1),
        W_prop, b_prop.reshape(1, -1))
    acc = _sc_edges(S, Dtab, smax[0],
                    edge_index[0].reshape(NC * NS * NCH, C),
                    edge_index[1].reshape(NC * NS * NCH, C))
    crossed = _post(acc, res, res_w.reshape(1, 1))
    return crossed, relo[0]
